# baseline (speedup 1.0000x reference)
import functools
import math

import jax
import jax.numpy as jnp
from jax import lax
from jax.experimental import pallas as pl
from jax.experimental.pallas import tpu as pltpu
from jax.experimental.pallas import tpu_sc as plsc

D_MODEL = 64
SCALE = math.sqrt(D_MODEL)

_info = plsc.get_sparse_core_info()
NC, NS, L = _info.num_cores, _info.num_subcores, _info.num_lanes
NW = NC * NS


def _make_kernel(B, D, C):
    per_w = B // NW
    nchunks = per_w // C
    mesh = plsc.VectorSubcoreMesh(core_axis_name="c", subcore_axis_name="s")

    @functools.partial(
        pl.kernel,
        mesh=mesh,
        out_type=jax.ShapeDtypeStruct((B, D), jnp.float32),
        scratch_types=[
            pltpu.VMEM((C,), jnp.int32),
            pltpu.VMEM((C, D), jnp.float32),
            pltpu.SemaphoreType.DMA,
        ],
        compiler_params=pltpu.CompilerParams(use_tc_tiling_on_sc=False),
    )
    def k(idx_hbm, lut_hbm, out_hbm, idx_v, rows_v, sem):
        wid = lax.axis_index("s") * NC + lax.axis_index("c")
        base = wid * per_w

        def chunk_body(g, carry):
            row0 = base + g * C
            pltpu.sync_copy(idx_hbm.at[pl.ds(row0, C)], idx_v)
            pltpu.async_copy(lut_hbm.at[idx_v], rows_v, sem).wait()

            def scale_row(r, c2):
                for j in range(D // L):
                    sl = pl.ds(j * L, L)
                    rows_v[r, sl] = rows_v[r, sl] * SCALE
                return c2

            lax.fori_loop(0, C, scale_row, 0)
            pltpu.sync_copy(rows_v, out_hbm.at[pl.ds(row0, C)])
            return carry

        lax.fori_loop(0, nchunks, chunk_body, 0)

    return k


def kernel(x, lut):
    B = x.shape[0] * x.shape[1]
    flat_idx = x.reshape(B)
    out = _make_kernel(B, D_MODEL, 512)(flat_idx, lut)
    return out.reshape(x.shape[0], x.shape[1], D_MODEL)


# speedup vs baseline: 1.1367x; 1.1367x over previous
import functools
import math

import jax
import jax.numpy as jnp
from jax import lax
from jax.experimental import pallas as pl
from jax.experimental.pallas import tpu as pltpu
from jax.experimental.pallas import tpu_sc as plsc

D_MODEL = 64
SCALE = math.sqrt(D_MODEL)

_info = plsc.get_sparse_core_info()
NC, NS, L = _info.num_cores, _info.num_subcores, _info.num_lanes
NW = NC * NS


def _make_kernel(B, D, C, NBUF, U):
    per_w = B // NW
    nchunks = per_w // C
    ngroups = nchunks // NBUF
    assert per_w % C == 0 and nchunks % NBUF == 0 and C % U == 0
    mesh = plsc.VectorSubcoreMesh(core_axis_name="c", subcore_axis_name="s")

    @functools.partial(
        pl.kernel,
        mesh=mesh,
        out_type=jax.ShapeDtypeStruct((B, D), jnp.float32),
        scratch_types=[
            pltpu.VMEM((NBUF, C), jnp.int32),
            pltpu.VMEM((NBUF, C, D), jnp.float32),
        ]
        + [pltpu.SemaphoreType.DMA] * (2 * NBUF),
        compiler_params=pltpu.CompilerParams(use_tc_tiling_on_sc=False),
    )
    def k(idx_hbm, lut_hbm, out_hbm, idx_v, rows_v, *sems):
        gsem, osem = sems[:NBUF], sems[NBUF:]
        wid = lax.axis_index("s") * NC + lax.axis_index("c")
        base = wid * per_w

        def scale_chunk(b):
            def body(r0, carry):
                for u in range(U):
                    r = r0 * U + u
                    for j in range(D // L):
                        sl = pl.ds(j * L, L)
                        rows_v[b, r, sl] = rows_v[b, r, sl] * SCALE
                return carry

            lax.fori_loop(0, C // U, body, 0)

        for b in range(NBUF):
            row0 = base + b * C
            pltpu.sync_copy(idx_hbm.at[pl.ds(row0, C)], idx_v.at[b])
            pltpu.async_copy(lut_hbm.at[idx_v.at[b]], rows_v.at[b], gsem[b])

        def group(gi, carry):
            for b in range(NBUF):
                g = gi * NBUF + b
                row0 = base + g * C
                pltpu.make_async_copy(
                    lut_hbm.at[idx_v.at[b]], rows_v.at[b], gsem[b]
                ).wait()
                scale_chunk(b)
                pltpu.async_copy(rows_v.at[b], out_hbm.at[pl.ds(row0, C)], osem[b])
                row0n = row0 + NBUF * C
                pltpu.sync_copy(idx_hbm.at[pl.ds(row0n, C)], idx_v.at[b])
                pltpu.make_async_copy(
                    rows_v.at[b], out_hbm.at[pl.ds(row0, C)], osem[b]
                ).wait()
                pltpu.async_copy(lut_hbm.at[idx_v.at[b]], rows_v.at[b], gsem[b])
            return carry

        lax.fori_loop(0, ngroups - 1, group, 0)

        for b in range(NBUF):
            g = (ngroups - 1) * NBUF + b
            row0 = base + g * C
            pltpu.make_async_copy(
                lut_hbm.at[idx_v.at[b]], rows_v.at[b], gsem[b]
            ).wait()
            scale_chunk(b)
            pltpu.async_copy(rows_v.at[b], out_hbm.at[pl.ds(row0, C)], osem[b])
        for b in range(NBUF):
            g = (ngroups - 1) * NBUF + b
            row0 = base + g * C
            pltpu.make_async_copy(
                rows_v.at[b], out_hbm.at[pl.ds(row0, C)], osem[b]
            ).wait()

    return k


def kernel(x, lut):
    B = x.shape[0] * x.shape[1]
    flat_idx = x.reshape(B)
    out = _make_kernel(B, D_MODEL, 640, 2, 8)(flat_idx, lut)
    return out.reshape(x.shape[0], x.shape[1], D_MODEL)
